# baseline (speedup 1.0000x reference)
import functools
import jax
import jax.numpy as jnp
from jax import lax
from jax.experimental import pallas as pl
from jax.experimental.pallas import tpu as pltpu
from jax.experimental.pallas import tpu_sc as plsc

N_FRAMES = 512
N_PIX = 128
N_CH = 3
BATCH = 64
PER_STEP = 8
STEPS = BATCH // PER_STEP

NC, NS = 2, 16
NW = NC * NS
B_PER_W = BATCH // NW


def _sc_gather(idx, Mmu, Mlv):
    mesh = plsc.VectorSubcoreMesh(core_axis_name="c", subcore_axis_name="s")

    @functools.partial(
        pl.kernel, mesh=mesh,
        out_type=[
            jax.ShapeDtypeStruct((BATCH, N_PIX, N_PIX), jnp.float32),
            jax.ShapeDtypeStruct((BATCH, N_PIX, N_PIX), jnp.float32),
        ],
        scratch_types=[
            pltpu.VMEM((B_PER_W,), jnp.int32),
            pltpu.VMEM((B_PER_W, N_PIX, N_PIX), jnp.float32),
            pltpu.SemaphoreType.DMA,
        ],
    )
    def k(idx_hbm, mmu_hbm, mlv_hbm, gmu_hbm, glv_hbm, idx_v, rows_v, sem):
        wid = lax.axis_index("s") * NC + lax.axis_index("c")
        base = wid * B_PER_W
        pltpu.sync_copy(idx_hbm.at[wid], idx_v)
        pltpu.async_copy(mmu_hbm.at[idx_v], rows_v, sem).wait()
        pltpu.sync_copy(rows_v, gmu_hbm.at[pl.ds(base, B_PER_W)])
        pltpu.async_copy(mlv_hbm.at[idx_v], rows_v, sem).wait()
        pltpu.sync_copy(rows_v, glv_hbm.at[pl.ds(base, B_PER_W)])

    return k(idx, Mmu, Mlv)


def _sig(x):
    return 0.5 + 0.5 * jnp.tanh(0.5 * x)


def _rtvf_body(idx_ref, gmu_ref, glv_ref, c_ref, *refs):
    b_ref, v_ref, ns_ref, nh_ref = refs[PER_STEP:PER_STEP + 4]
    out_ref = refs[-1]
    step = pl.program_id(0)

    for k in range(PER_STEP):
        h_ref = refs[k]
        f = idx_ref[step * PER_STEP + k]
        c = c_ref[0, f]

        th = jnp.tanh(0.5 * gmu_ref[k])
        a = 0.5 - 0.5 * th
        one_m_a = 0.5 + 0.5 * th
        e = jnp.exp(glv_ref[k])
        ns = ns_ref[k]
        nh = nh_ref[k]

        for ch in range(N_CH):
            s = b_ref[ch] + c * v_ref[0, ch]
            hc = h_ref[0, ch]
            t = a * s + one_m_a * hc + e * (s * s * ns + hc * hc * nh)
            out_ref[k, ch] = _sig(t)


def kernel(index, img, B, V, C, Mmu, Mlv, H, noise_S, noise_H):
    del img
    idx = index.astype(jnp.int32)
    ht = jnp.transpose(H, (0, 3, 1, 2))
    vt = jnp.transpose(V, (0, 3, 1, 2))
    bt = jnp.transpose(B, (2, 0, 1))
    ct = jnp.transpose(C, (1, 0))
    ns = noise_S.reshape(BATCH, N_PIX, N_PIX)
    nh = noise_H.reshape(BATCH, N_PIX, N_PIX)

    gmu, glv = _sc_gather(idx.reshape(NW, B_PER_W), Mmu, Mlv)

    def gspec4(k):
        return lambda b, i: (i[b * PER_STEP + k], 0, 0, 0)

    in_specs = [
        pl.BlockSpec((PER_STEP, N_PIX, N_PIX), lambda b, i: (b, 0, 0)),
        pl.BlockSpec((PER_STEP, N_PIX, N_PIX), lambda b, i: (b, 0, 0)),
        pl.BlockSpec(memory_space=pltpu.SMEM),
    ]
    operands = [gmu, glv, ct]
    for k in range(PER_STEP):
        in_specs.append(pl.BlockSpec((1, N_CH, N_PIX, N_PIX), gspec4(k)))
        operands.append(ht)
    in_specs.extend([
        pl.BlockSpec((N_CH, N_PIX, N_PIX), lambda b, i: (0, 0, 0)),
        pl.BlockSpec((1, N_CH, N_PIX, N_PIX),
                     lambda b, i: (0, 0, 0, 0)),
        pl.BlockSpec((PER_STEP, N_PIX, N_PIX), lambda b, i: (b, 0, 0)),
        pl.BlockSpec((PER_STEP, N_PIX, N_PIX), lambda b, i: (b, 0, 0)),
    ])
    operands.extend([bt, vt, ns, nh])

    grid_spec = pltpu.PrefetchScalarGridSpec(
        num_scalar_prefetch=1,
        grid=(STEPS,),
        in_specs=in_specs,
        out_specs=pl.BlockSpec((PER_STEP, N_CH, N_PIX, N_PIX),
                               lambda b, i: (b, 0, 0, 0)),
    )

    out = pl.pallas_call(
        _rtvf_body,
        grid_spec=grid_spec,
        out_shape=jax.ShapeDtypeStruct((BATCH, N_CH, N_PIX, N_PIX),
                                       jnp.float32),
        compiler_params=pltpu.CompilerParams(
            dimension_semantics=("arbitrary",),
        ),
    )(idx, *operands)

    return jnp.transpose(out, (0, 2, 3, 1))


# speedup vs baseline: 2.4658x; 2.4658x over previous
import jax
import jax.numpy as jnp
from jax.experimental import pallas as pl
from jax.experimental.pallas import tpu as pltpu

N_FRAMES = 512
N_PIX = 128
N_CH = 3
BATCH = 64
PER_STEP = 16
STEPS = BATCH // PER_STEP


def _sig(x):
    return 0.5 + 0.5 * jnp.tanh(0.5 * x)


def _rtvf_body(idx_ref, *refs):
    c_ref, b_ref, v_ref, ns_ref, nh_ref = refs[3 * PER_STEP:3 * PER_STEP + 5]
    out_ref = refs[-1]
    step = pl.program_id(0)

    for k in range(PER_STEP):
        mmu_ref, mlv_ref, h_ref = refs[3 * k:3 * k + 3]
        f = idx_ref[step * PER_STEP + k]
        c = c_ref[0, f]

        th = jnp.tanh(0.5 * mmu_ref[0])
        a = 0.5 - 0.5 * th
        e = jnp.exp(mlv_ref[0])
        es = e * ns_ref[k]
        eh = e * nh_ref[k]

        for ch in range(N_CH):
            s = b_ref[ch] + c * v_ref[0, ch]
            hc = h_ref[0, ch]
            t = a * (s - hc) + hc + s * s * es + hc * hc * eh
            out_ref[k, ch] = _sig(t)


def kernel(index, img, B, V, C, Mmu, Mlv, H, noise_S, noise_H):
    del img
    idx = index.astype(jnp.int32)
    ht = jnp.transpose(H, (0, 3, 1, 2))
    vt = jnp.transpose(V, (0, 3, 1, 2))
    bt = jnp.transpose(B, (2, 0, 1))
    ct = jnp.transpose(C, (1, 0))
    ns = noise_S.reshape(BATCH, N_PIX, N_PIX)
    nh = noise_H.reshape(BATCH, N_PIX, N_PIX)

    def gspec(k):
        return lambda b, i: (i[b * PER_STEP + k], 0, 0)

    def gspec4(k):
        return lambda b, i: (i[b * PER_STEP + k], 0, 0, 0)

    in_specs = []
    operands = []
    for k in range(PER_STEP):
        in_specs.append(pl.BlockSpec((1, N_PIX, N_PIX), gspec(k)))
        in_specs.append(pl.BlockSpec((1, N_PIX, N_PIX), gspec(k)))
        in_specs.append(pl.BlockSpec((1, N_CH, N_PIX, N_PIX), gspec4(k)))
        operands.extend([Mmu, Mlv, ht])
    in_specs.extend([
        pl.BlockSpec(memory_space=pltpu.SMEM),
        pl.BlockSpec((N_CH, N_PIX, N_PIX), lambda b, i: (0, 0, 0)),
        pl.BlockSpec((1, N_CH, N_PIX, N_PIX),
                     lambda b, i: (0, 0, 0, 0)),
        pl.BlockSpec((PER_STEP, N_PIX, N_PIX), lambda b, i: (b, 0, 0)),
        pl.BlockSpec((PER_STEP, N_PIX, N_PIX), lambda b, i: (b, 0, 0)),
    ])
    operands.extend([ct, bt, vt, ns, nh])

    grid_spec = pltpu.PrefetchScalarGridSpec(
        num_scalar_prefetch=1,
        grid=(STEPS,),
        in_specs=in_specs,
        out_specs=pl.BlockSpec((PER_STEP, N_CH, N_PIX, N_PIX),
                               lambda b, i: (b, 0, 0, 0)),
    )

    out = pl.pallas_call(
        _rtvf_body,
        grid_spec=grid_spec,
        out_shape=jax.ShapeDtypeStruct((BATCH, N_CH, N_PIX, N_PIX),
                                       jnp.float32),
        compiler_params=pltpu.CompilerParams(
            dimension_semantics=("arbitrary",),
        ),
    )(idx, *operands)

    return jnp.transpose(out, (0, 2, 3, 1))
